# baseline (speedup 1.0000x reference)
import functools

import jax
import jax.numpy as jnp
from jax import lax
from jax.experimental import pallas as pl
from jax.experimental.pallas import tpu as pltpu
from jax.experimental.pallas import tpu_sc as plsc

NC = 2
NS = 16
NW = NC * NS
EW = 128

_HI = jax.lax.Precision.HIGHEST


def _make_sc_segment_sum(n, d, k, n_pad):
  mesh = plsc.VectorSubcoreMesh(core_axis_name="c", subcore_axis_name="s")
  rpt = (-(-n // NS) + 7) // 8 * 8
  rpt_last = n - (NS - 1) * rpt

  nbuf = 2
  iw = 40

  @functools.partial(
      pl.kernel,
      out_type=jax.ShapeDtypeStruct((2 * n, d), jnp.float32),
      mesh=mesh,
      scratch_types=(
          [pltpu.VMEM((iw, EW), jnp.int32),
           pltpu.VMEM((iw, EW), jnp.int32)]
          + [pltpu.VMEM((EW, d), jnp.float32) for _ in range(nbuf)]
          + [pltpu.VMEM_SHARED((n_pad, d), jnp.float32)]
          + [pltpu.SemaphoreType.DMA for _ in range(2 * nbuf + 1)]
      ),
  )
  def seg_sum(x_hbm, ei_hbm, out_hbm, src_v, dst_v,
              r0, r1, acc_sh, g0, g1, s0, s1, seedsem):
    rows = (r0, r1)
    gsem = (g0, g1)
    ssem = (s0, s1)
    c = lax.axis_index("c")
    s = lax.axis_index("s")
    w = c * NS + s
    base = s * rpt

    @pl.when(s < NS - 1)
    def _():
      pltpu.async_copy(x_hbm.at[pl.ds(base, rpt)],
                       acc_sh.at[pl.ds(base, rpt)], seedsem)

    @pl.when(s == NS - 1)
    def _():
      pltpu.async_copy(x_hbm.at[pl.ds(base, rpt_last)],
                       acc_sh.at[pl.ds(base, rpt_last)], seedsem)

    pltpu.sync_copy(ei_hbm.at[0, w, pl.ds(0, iw)], src_v)
    pltpu.sync_copy(ei_hbm.at[1, w, pl.ds(0, iw)], dst_v)
    for b in range(nbuf):
      pltpu.async_copy(x_hbm.at[src_v.at[b]], rows[b], gsem[b])

    @pl.when(s < NS - 1)
    def _():
      pltpu.make_async_copy(x_hbm.at[pl.ds(base, rpt)],
                            acc_sh.at[pl.ds(base, rpt)], seedsem).wait()

    @pl.when(s == NS - 1)
    def _():
      pltpu.make_async_copy(x_hbm.at[pl.ds(base, rpt_last)],
                            acc_sh.at[pl.ds(base, rpt_last)], seedsem).wait()

    plsc.subcore_barrier()

    @pl.loop(0, k // iw)
    def _(wi):
      @pl.when(wi > 0)
      def _():
        pltpu.sync_copy(ei_hbm.at[0, w, pl.ds(wi * iw, iw)], src_v)
        pltpu.sync_copy(ei_hbm.at[1, w, pl.ds(wi * iw, iw)], dst_v)
        for b in range(nbuf):
          pltpu.async_copy(x_hbm.at[src_v.at[b]], rows[b], gsem[b])

      @pl.loop(0, iw // nbuf)
      def _(i):
        j0 = i * nbuf
        for b in range(nbuf):
          pltpu.make_async_copy(x_hbm.at[src_v.at[j0 + b]], rows[b],
                                gsem[b]).wait()
          pltpu.async_copy(rows[b], acc_sh.at[dst_v.at[j0 + b]], ssem[b],
                           add=True)
        for b in range(nbuf):
          pltpu.make_async_copy(rows[b], acc_sh.at[dst_v.at[j0 + b]],
                                ssem[b]).wait()

          @pl.when(j0 + b + nbuf < iw)
          def _():
            pltpu.async_copy(x_hbm.at[src_v.at[j0 + b + nbuf]], rows[b],
                             gsem[b])

    plsc.subcore_barrier()

    @pl.when(s < NS - 1)
    def _():
      pltpu.sync_copy(acc_sh.at[pl.ds(base, rpt)],
                      out_hbm.at[pl.ds(c * n + base, rpt)])

    @pl.when(s == NS - 1)
    def _():
      pltpu.sync_copy(acc_sh.at[pl.ds(base, rpt_last)],
                      out_hbm.at[pl.ds(c * n + base, rpt_last)])

  return seg_sum


def _dot(a, b):
  return lax.dot_general(a.astype(jnp.bfloat16), b.astype(jnp.bfloat16),
                         (((1,), (0,)), ((), ())),
                         preferred_element_type=jnp.float32)


def _mlp_stack(u, w0, b0, w1, b1, w2, b2):
  h = jnp.tanh(_dot(u, w0) + b0)
  h = jnp.tanh(_dot(h, w1) + b1)
  h = jnp.tanh(_dot(h, w2) + b2)
  return jnp.tanh(h)


def _make_mlp(n, d, h, r):
  grid = n // r

  def body(p0_ref, p1_ref, x_ref, w0_ref, b0_ref, w1_ref, b1_ref, w2_ref,
           b2_ref, o_ref):
    u = p0_ref[...] + p1_ref[...] - x_ref[...]
    o_ref[...] = _mlp_stack(u, w0_ref[...], b0_ref[...], w1_ref[...],
                            b1_ref[...], w2_ref[...], b2_ref[...])

  row_spec = pl.BlockSpec((r, d), lambda i: (i, 0))
  return pl.pallas_call(
      body,
      grid=(grid,),
      in_specs=[
          row_spec, pl.BlockSpec((r, d), lambda i: (i + grid, 0)), row_spec,
          pl.BlockSpec((d, h), lambda i: (0, 0)),
          pl.BlockSpec((1, h), lambda i: (0, 0)),
          pl.BlockSpec((h, h), lambda i: (0, 0)),
          pl.BlockSpec((1, h), lambda i: (0, 0)),
          pl.BlockSpec((h, d), lambda i: (0, 0)),
          pl.BlockSpec((1, d), lambda i: (0, 0)),
      ],
      out_specs=row_spec,
      out_shape=jax.ShapeDtypeStruct((n, d), jnp.float32),
  )


def _make_mlp_readout(n, d, h, r, g):
  grid = n // r

  def body(p0_ref, p1_ref, x_ref, w0_ref, b0_ref, w1_ref, b1_ref, w2_ref,
           b2_ref, batch_ref, o_ref, sums_ref, counts_ref):
    i = pl.program_id(0)

    @pl.when(i == 0)
    def _():
      sums_ref[...] = jnp.zeros_like(sums_ref)
      counts_ref[...] = jnp.zeros_like(counts_ref)

    u = p0_ref[...] + p1_ref[...] - x_ref[...]
    xn = _mlp_stack(u, w0_ref[...], b0_ref[...], w1_ref[...], b1_ref[...],
                    w2_ref[...], b2_ref[...])
    gids = lax.broadcasted_iota(jnp.int32, (g, r), 0)
    onehot = (gids == batch_ref[0]).astype(jnp.float32)
    sums_ref[...] += lax.dot_general(
        onehot, xn, (((1,), (0,)), ((), ())),
        precision=_HI, preferred_element_type=jnp.float32)
    cnt = jnp.sum(onehot, axis=1, keepdims=True)
    counts_ref[...] += jnp.broadcast_to(cnt, (g, d))

    @pl.when(i == grid - 1)
    def _():
      o_ref[...] = sums_ref[...] / jnp.maximum(counts_ref[...], 1.0)

  row_spec = pl.BlockSpec((r, d), lambda i: (i, 0))
  return pl.pallas_call(
      body,
      grid=(grid,),
      in_specs=[
          row_spec, pl.BlockSpec((r, d), lambda i: (i + grid, 0)), row_spec,
          pl.BlockSpec((d, h), lambda i: (0, 0)),
          pl.BlockSpec((1, h), lambda i: (0, 0)),
          pl.BlockSpec((h, h), lambda i: (0, 0)),
          pl.BlockSpec((1, h), lambda i: (0, 0)),
          pl.BlockSpec((h, d), lambda i: (0, 0)),
          pl.BlockSpec((1, d), lambda i: (0, 0)),
          pl.BlockSpec((1, 1, r), lambda i: (i, 0, 0)),
      ],
      out_specs=pl.BlockSpec((g, d), lambda i: (0, 0)),
      out_shape=jax.ShapeDtypeStruct((g, d), jnp.float32),
      scratch_shapes=[
          pltpu.VMEM((g, d), jnp.float32),
          pltpu.VMEM((g, d), jnp.float32),
      ],
  )


def kernel(attrs, edge_index, batch,
           W0_0, b0_0, W0_1, b0_1, W0_2, b0_2,
           W1_0, b1_0, W1_1, b1_1, W1_2, b1_2):
  n, d = attrs.shape
  e = edge_index.shape[1]
  h = W0_0.shape[1]
  g = 64
  r = 2000
  k = -(-e // (NW * EW))
  k = -(-k // 40) * 40
  e_pad = NW * k * EW
  n_pad = n + EW

  pad = e_pad - e
  pad_iota = jnp.arange(pad, dtype=jnp.int32) % EW
  ei4d = jnp.concatenate(
      [edge_index, jnp.stack([pad_iota, n + pad_iota])], axis=1
  ).reshape(2, NW, k, EW)
  batch3d = batch.reshape(n // r, 1, r)
  b0s = (b0_0.reshape(1, h), b0_1.reshape(1, h), b0_2.reshape(1, d))
  b1s = (b1_0.reshape(1, h), b1_1.reshape(1, h), b1_2.reshape(1, d))

  seg_sum = _make_sc_segment_sum(n, d, k, n_pad)
  mlp = _make_mlp(n, d, h, r)
  mlp_readout = _make_mlp_readout(n, d, h, r, g)

  p = seg_sum(attrs, ei4d)
  x1 = mlp(p, p, attrs, W0_0, b0s[0], W0_1, b0s[1], W0_2, b0s[2])
  q = seg_sum(x1, ei4d)
  out = mlp_readout(q, q, x1, W1_0, b1s[0], W1_1, b1s[1],
                    W1_2, b1s[2], batch3d)
  return out


# speedup vs baseline: 1.0001x; 1.0001x over previous
import functools

import jax
import jax.numpy as jnp
from jax import lax
from jax.experimental import pallas as pl
from jax.experimental.pallas import tpu as pltpu
from jax.experimental.pallas import tpu_sc as plsc

NC = 2
NS = 16
NW = NC * NS
EW = 128

_HI = jax.lax.Precision.HIGHEST


def _make_sc_segment_sum(n, d, k, n_pad):
  mesh = plsc.VectorSubcoreMesh(core_axis_name="c", subcore_axis_name="s")
  rpt = (-(-n // NS) + 7) // 8 * 8
  rpt_last = n - (NS - 1) * rpt

  nbuf = 2
  iw = 40

  @functools.partial(
      pl.kernel,
      out_type=jax.ShapeDtypeStruct((2 * n, d), jnp.float32),
      mesh=mesh,
      scratch_types=(
          [pltpu.VMEM((iw, EW), jnp.int32),
           pltpu.VMEM((iw, EW), jnp.int32)]
          + [pltpu.VMEM((EW, d), jnp.float32) for _ in range(nbuf)]
          + [pltpu.VMEM_SHARED((n_pad, d), jnp.float32)]
          + [pltpu.SemaphoreType.DMA for _ in range(2 * nbuf + 1)]
      ),
  )
  def seg_sum(x_hbm, ei_hbm, out_hbm, src_v, dst_v,
              r0, r1, acc_sh, g0, g1, s0, s1, seedsem):
    rows = (r0, r1)
    gsem = (g0, g1)
    ssem = (s0, s1)
    c = lax.axis_index("c")
    s = lax.axis_index("s")
    w = c * NS + s
    base = s * rpt

    @pl.when(s < NS - 1)
    def _():
      pltpu.async_copy(x_hbm.at[pl.ds(base, rpt)],
                       acc_sh.at[pl.ds(base, rpt)], seedsem)

    @pl.when(s == NS - 1)
    def _():
      pltpu.async_copy(x_hbm.at[pl.ds(base, rpt_last)],
                       acc_sh.at[pl.ds(base, rpt_last)], seedsem)

    pltpu.sync_copy(ei_hbm.at[0, w, pl.ds(0, iw)], src_v)
    pltpu.sync_copy(ei_hbm.at[1, w, pl.ds(0, iw)], dst_v)
    for b in range(nbuf):
      pltpu.async_copy(x_hbm.at[src_v.at[b]], rows[b], gsem[b])

    @pl.when(s < NS - 1)
    def _():
      pltpu.make_async_copy(x_hbm.at[pl.ds(base, rpt)],
                            acc_sh.at[pl.ds(base, rpt)], seedsem).wait()

    @pl.when(s == NS - 1)
    def _():
      pltpu.make_async_copy(x_hbm.at[pl.ds(base, rpt_last)],
                            acc_sh.at[pl.ds(base, rpt_last)], seedsem).wait()

    plsc.subcore_barrier()

    @pl.loop(0, k // iw)
    def _(wi):
      @pl.when(wi > 0)
      def _():
        pltpu.sync_copy(ei_hbm.at[0, w, pl.ds(wi * iw, iw)], src_v)
        pltpu.sync_copy(ei_hbm.at[1, w, pl.ds(wi * iw, iw)], dst_v)
        for b in range(nbuf):
          pltpu.async_copy(x_hbm.at[src_v.at[b]], rows[b], gsem[b])

      @pl.loop(0, iw // nbuf)
      def _(i):
        j0 = i * nbuf
        for b in range(nbuf):
          pltpu.make_async_copy(x_hbm.at[src_v.at[j0 + b]], rows[b],
                                gsem[b]).wait()
          pltpu.async_copy(rows[b], acc_sh.at[dst_v.at[j0 + b]], ssem[b],
                           add=True)
        for b in range(nbuf):
          pltpu.make_async_copy(rows[b], acc_sh.at[dst_v.at[j0 + b]],
                                ssem[b]).wait()

          @pl.when(j0 + b + nbuf < iw)
          def _():
            pltpu.async_copy(x_hbm.at[src_v.at[j0 + b + nbuf]], rows[b],
                             gsem[b])

    plsc.subcore_barrier()

    @pl.when(s < NS - 1)
    def _():
      pltpu.sync_copy(acc_sh.at[pl.ds(base, rpt)],
                      out_hbm.at[pl.ds(c * n + base, rpt)])

    @pl.when(s == NS - 1)
    def _():
      pltpu.sync_copy(acc_sh.at[pl.ds(base, rpt_last)],
                      out_hbm.at[pl.ds(c * n + base, rpt_last)])

  return seg_sum


def _dot(a, b):
  return lax.dot_general(a, b, (((1,), (0,)), ((), ())),
                         precision=jax.lax.Precision.DEFAULT,
                         preferred_element_type=jnp.float32)


def _mlp_stack(u, w0, b0, w1, b1, w2, b2):
  h = jnp.tanh(_dot(u, w0) + b0)
  h = jnp.tanh(_dot(h, w1) + b1)
  h = jnp.tanh(_dot(h, w2) + b2)
  return jnp.tanh(h)


def _make_mlp(n, d, h, r):
  grid = n // r

  def body(p0_ref, p1_ref, x_ref, w0_ref, b0_ref, w1_ref, b1_ref, w2_ref,
           b2_ref, o_ref):
    u = p0_ref[...] + p1_ref[...] - x_ref[...]
    o_ref[...] = _mlp_stack(u, w0_ref[...], b0_ref[...], w1_ref[...],
                            b1_ref[...], w2_ref[...], b2_ref[...])

  row_spec = pl.BlockSpec((r, d), lambda i: (i, 0))
  return pl.pallas_call(
      body,
      grid=(grid,),
      in_specs=[
          row_spec, pl.BlockSpec((r, d), lambda i: (i + grid, 0)), row_spec,
          pl.BlockSpec((d, h), lambda i: (0, 0)),
          pl.BlockSpec((1, h), lambda i: (0, 0)),
          pl.BlockSpec((h, h), lambda i: (0, 0)),
          pl.BlockSpec((1, h), lambda i: (0, 0)),
          pl.BlockSpec((h, d), lambda i: (0, 0)),
          pl.BlockSpec((1, d), lambda i: (0, 0)),
      ],
      out_specs=row_spec,
      out_shape=jax.ShapeDtypeStruct((n, d), jnp.float32),
  )


def _make_mlp_readout(n, d, h, r, g):
  grid = n // r

  def body(p0_ref, p1_ref, x_ref, w0_ref, b0_ref, w1_ref, b1_ref, w2_ref,
           b2_ref, batch_ref, o_ref, sums_ref, counts_ref):
    i = pl.program_id(0)

    @pl.when(i == 0)
    def _():
      sums_ref[...] = jnp.zeros_like(sums_ref)
      counts_ref[...] = jnp.zeros_like(counts_ref)

    u = p0_ref[...] + p1_ref[...] - x_ref[...]
    xn = _mlp_stack(u, w0_ref[...], b0_ref[...], w1_ref[...], b1_ref[...],
                    w2_ref[...], b2_ref[...])
    gids = lax.broadcasted_iota(jnp.int32, (g, r), 0)
    onehot = (gids == batch_ref[0]).astype(jnp.float32)
    sums_ref[...] += lax.dot_general(
        onehot, xn, (((1,), (0,)), ((), ())),
        precision=_HI, preferred_element_type=jnp.float32)
    cnt = jnp.sum(onehot, axis=1, keepdims=True)
    counts_ref[...] += jnp.broadcast_to(cnt, (g, d))

    @pl.when(i == grid - 1)
    def _():
      o_ref[...] = sums_ref[...] / jnp.maximum(counts_ref[...], 1.0)

  row_spec = pl.BlockSpec((r, d), lambda i: (i, 0))
  return pl.pallas_call(
      body,
      grid=(grid,),
      in_specs=[
          row_spec, pl.BlockSpec((r, d), lambda i: (i + grid, 0)), row_spec,
          pl.BlockSpec((d, h), lambda i: (0, 0)),
          pl.BlockSpec((1, h), lambda i: (0, 0)),
          pl.BlockSpec((h, h), lambda i: (0, 0)),
          pl.BlockSpec((1, h), lambda i: (0, 0)),
          pl.BlockSpec((h, d), lambda i: (0, 0)),
          pl.BlockSpec((1, d), lambda i: (0, 0)),
          pl.BlockSpec((1, 1, r), lambda i: (i, 0, 0)),
      ],
      out_specs=pl.BlockSpec((g, d), lambda i: (0, 0)),
      out_shape=jax.ShapeDtypeStruct((g, d), jnp.float32),
      scratch_shapes=[
          pltpu.VMEM((g, d), jnp.float32),
          pltpu.VMEM((g, d), jnp.float32),
      ],
  )


def kernel(attrs, edge_index, batch,
           W0_0, b0_0, W0_1, b0_1, W0_2, b0_2,
           W1_0, b1_0, W1_1, b1_1, W1_2, b1_2):
  n, d = attrs.shape
  e = edge_index.shape[1]
  h = W0_0.shape[1]
  g = 64
  r = 2000
  k = -(-e // (NW * EW))
  k = -(-k // 40) * 40
  e_pad = NW * k * EW
  n_pad = n + EW

  pad = e_pad - e
  pad_iota = jnp.arange(pad, dtype=jnp.int32) % EW
  ei4d = jnp.concatenate(
      [edge_index, jnp.stack([pad_iota, n + pad_iota])], axis=1
  ).reshape(2, NW, k, EW)
  batch3d = batch.reshape(n // r, 1, r)
  b0s = (b0_0.reshape(1, h), b0_1.reshape(1, h), b0_2.reshape(1, d))
  b1s = (b1_0.reshape(1, h), b1_1.reshape(1, h), b1_2.reshape(1, d))

  seg_sum = _make_sc_segment_sum(n, d, k, n_pad)
  mlp = _make_mlp(n, d, h, r)
  mlp_readout = _make_mlp_readout(n, d, h, r, g)

  p = seg_sum(attrs, ei4d)
  x1 = mlp(p, p, attrs, W0_0, b0s[0], W0_1, b0s[1], W0_2, b0s[2])
  q = seg_sum(x1, ei4d)
  out = mlp_readout(q, q, x1, W1_0, b1s[0], W1_1, b1s[1],
                    W1_2, b1s[2], batch3d)
  return out


# speedup vs baseline: 1.0018x; 1.0017x over previous
import functools

import jax
import jax.numpy as jnp
from jax import lax
from jax.experimental import pallas as pl
from jax.experimental.pallas import tpu as pltpu
from jax.experimental.pallas import tpu_sc as plsc

NC = 2
NS = 16
NW = NC * NS
EW = 128


def _make_sc_segment_sum(n, d, k, n_pad):
  mesh = plsc.VectorSubcoreMesh(core_axis_name="c", subcore_axis_name="s")
  rpt = (-(-n // NS) + 7) // 8 * 8
  rpt_last = n - (NS - 1) * rpt

  nbuf = 2
  iw = 40

  @functools.partial(
      pl.kernel,
      out_type=jax.ShapeDtypeStruct((2 * n, d), jnp.float32),
      mesh=mesh,
      scratch_types=(
          [pltpu.VMEM((iw, EW), jnp.int32),
           pltpu.VMEM((iw, EW), jnp.int32)]
          + [pltpu.VMEM((EW, d), jnp.float32) for _ in range(nbuf)]
          + [pltpu.VMEM_SHARED((n_pad, d), jnp.float32)]
          + [pltpu.SemaphoreType.DMA for _ in range(2 * nbuf + 1)]
      ),
  )
  def seg_sum(x_hbm, ei_hbm, out_hbm, src_v, dst_v,
              r0, r1, acc_sh, g0, g1, s0, s1, seedsem):
    rows = (r0, r1)
    gsem = (g0, g1)
    ssem = (s0, s1)
    c = lax.axis_index("c")
    s = lax.axis_index("s")
    w = c * NS + s
    base = s * rpt

    @pl.when(s < NS - 1)
    def _():
      pltpu.async_copy(x_hbm.at[pl.ds(base, rpt)],
                       acc_sh.at[pl.ds(base, rpt)], seedsem)

    @pl.when(s == NS - 1)
    def _():
      pltpu.async_copy(x_hbm.at[pl.ds(base, rpt_last)],
                       acc_sh.at[pl.ds(base, rpt_last)], seedsem)

    pltpu.sync_copy(ei_hbm.at[0, w, pl.ds(0, iw)], src_v)
    pltpu.sync_copy(ei_hbm.at[1, w, pl.ds(0, iw)], dst_v)
    for b in range(nbuf):
      pltpu.async_copy(x_hbm.at[src_v.at[b]], rows[b], gsem[b])

    @pl.when(s < NS - 1)
    def _():
      pltpu.make_async_copy(x_hbm.at[pl.ds(base, rpt)],
                            acc_sh.at[pl.ds(base, rpt)], seedsem).wait()

    @pl.when(s == NS - 1)
    def _():
      pltpu.make_async_copy(x_hbm.at[pl.ds(base, rpt_last)],
                            acc_sh.at[pl.ds(base, rpt_last)], seedsem).wait()

    plsc.subcore_barrier()

    @pl.loop(0, k // iw)
    def _(wi):
      @pl.when(wi > 0)
      def _():
        pltpu.sync_copy(ei_hbm.at[0, w, pl.ds(wi * iw, iw)], src_v)
        pltpu.sync_copy(ei_hbm.at[1, w, pl.ds(wi * iw, iw)], dst_v)
        for b in range(nbuf):
          pltpu.async_copy(x_hbm.at[src_v.at[b]], rows[b], gsem[b])

      @pl.loop(0, iw // nbuf)
      def _(i):
        j0 = i * nbuf
        for b in range(nbuf):
          pltpu.make_async_copy(x_hbm.at[src_v.at[j0 + b]], rows[b],
                                gsem[b]).wait()
          pltpu.async_copy(rows[b], acc_sh.at[dst_v.at[j0 + b]], ssem[b],
                           add=True)
        for b in range(nbuf):
          pltpu.make_async_copy(rows[b], acc_sh.at[dst_v.at[j0 + b]],
                                ssem[b]).wait()

          @pl.when(j0 + b + nbuf < iw)
          def _():
            pltpu.async_copy(x_hbm.at[src_v.at[j0 + b + nbuf]], rows[b],
                             gsem[b])

    plsc.subcore_barrier()

    @pl.when(s < NS - 1)
    def _():
      pltpu.sync_copy(acc_sh.at[pl.ds(base, rpt)],
                      out_hbm.at[pl.ds(c * n + base, rpt)])

    @pl.when(s == NS - 1)
    def _():
      pltpu.sync_copy(acc_sh.at[pl.ds(base, rpt_last)],
                      out_hbm.at[pl.ds(c * n + base, rpt_last)])

  return seg_sum


def _dot(a, b):
  return lax.dot_general(a, b, (((1,), (0,)), ((), ())),
                         precision=jax.lax.Precision.DEFAULT,
                         preferred_element_type=jnp.float32)


def _mlp_stack(u, w0, b0, w1, b1, w2, b2):
  h = jnp.tanh(_dot(u, w0) + b0)
  h = jnp.tanh(_dot(h, w1) + b1)
  h = jnp.tanh(_dot(h, w2) + b2)
  return jnp.tanh(h)


def _make_mlp(n, d, h, r):
  grid = n // r

  def body(p0_ref, p1_ref, x_ref, w0_ref, b0_ref, w1_ref, b1_ref, w2_ref,
           b2_ref, o_ref):
    u = p0_ref[...] + p1_ref[...] - x_ref[...]
    o_ref[...] = _mlp_stack(u, w0_ref[...], b0_ref[...], w1_ref[...],
                            b1_ref[...], w2_ref[...], b2_ref[...])

  row_spec = pl.BlockSpec((r, d), lambda i: (i, 0))
  return pl.pallas_call(
      body,
      grid=(grid,),
      in_specs=[
          row_spec, pl.BlockSpec((r, d), lambda i: (i + grid, 0)), row_spec,
          pl.BlockSpec((d, h), lambda i: (0, 0)),
          pl.BlockSpec((1, h), lambda i: (0, 0)),
          pl.BlockSpec((h, h), lambda i: (0, 0)),
          pl.BlockSpec((1, h), lambda i: (0, 0)),
          pl.BlockSpec((h, d), lambda i: (0, 0)),
          pl.BlockSpec((1, d), lambda i: (0, 0)),
      ],
      out_specs=row_spec,
      out_shape=jax.ShapeDtypeStruct((n, d), jnp.float32),
  )


def _make_mlp_readout(n, d, h, r, g):
  grid = n // r

  def body(p0_ref, p1_ref, x_ref, w0_ref, b0_ref, w1_ref, b1_ref, w2_ref,
           b2_ref, batch_ref, o_ref, sums_ref, counts_ref):
    i = pl.program_id(0)

    @pl.when(i == 0)
    def _():
      sums_ref[...] = jnp.zeros_like(sums_ref)
      counts_ref[...] = jnp.zeros_like(counts_ref)

    u = p0_ref[...] + p1_ref[...] - x_ref[...]
    xn = _mlp_stack(u, w0_ref[...], b0_ref[...], w1_ref[...], b1_ref[...],
                    w2_ref[...], b2_ref[...])
    gids = lax.broadcasted_iota(jnp.int32, (g, r), 0)
    onehot = (gids == batch_ref[0]).astype(jnp.float32)
    sums_ref[...] += _dot(onehot, xn)
    cnt = jnp.sum(onehot, axis=1, keepdims=True)
    counts_ref[...] += jnp.broadcast_to(cnt, (g, d))

    @pl.when(i == grid - 1)
    def _():
      o_ref[...] = sums_ref[...] / jnp.maximum(counts_ref[...], 1.0)

  row_spec = pl.BlockSpec((r, d), lambda i: (i, 0))
  return pl.pallas_call(
      body,
      grid=(grid,),
      in_specs=[
          row_spec, pl.BlockSpec((r, d), lambda i: (i + grid, 0)), row_spec,
          pl.BlockSpec((d, h), lambda i: (0, 0)),
          pl.BlockSpec((1, h), lambda i: (0, 0)),
          pl.BlockSpec((h, h), lambda i: (0, 0)),
          pl.BlockSpec((1, h), lambda i: (0, 0)),
          pl.BlockSpec((h, d), lambda i: (0, 0)),
          pl.BlockSpec((1, d), lambda i: (0, 0)),
          pl.BlockSpec((1, 1, r), lambda i: (i, 0, 0)),
      ],
      out_specs=pl.BlockSpec((g, d), lambda i: (0, 0)),
      out_shape=jax.ShapeDtypeStruct((g, d), jnp.float32),
      scratch_shapes=[
          pltpu.VMEM((g, d), jnp.float32),
          pltpu.VMEM((g, d), jnp.float32),
      ],
  )


def kernel(attrs, edge_index, batch,
           W0_0, b0_0, W0_1, b0_1, W0_2, b0_2,
           W1_0, b1_0, W1_1, b1_1, W1_2, b1_2):
  n, d = attrs.shape
  e = edge_index.shape[1]
  h = W0_0.shape[1]
  g = 64
  r = 2000
  k = -(-e // (NW * EW))
  k = -(-k // 40) * 40
  e_pad = NW * k * EW
  n_pad = n + EW

  pad = e_pad - e
  pad_iota = jnp.arange(pad, dtype=jnp.int32) % EW
  ei4d = jnp.concatenate(
      [edge_index, jnp.stack([pad_iota, n + pad_iota])], axis=1
  ).reshape(2, NW, k, EW)
  batch3d = batch.reshape(n // r, 1, r)
  b0s = (b0_0.reshape(1, h), b0_1.reshape(1, h), b0_2.reshape(1, d))
  b1s = (b1_0.reshape(1, h), b1_1.reshape(1, h), b1_2.reshape(1, d))

  seg_sum = _make_sc_segment_sum(n, d, k, n_pad)
  mlp = _make_mlp(n, d, h, r)
  mlp_readout = _make_mlp_readout(n, d, h, r, g)

  p = seg_sum(attrs, ei4d)
  x1 = mlp(p, p, attrs, W0_0, b0s[0], W0_1, b0s[1], W0_2, b0s[2])
  q = seg_sum(x1, ei4d)
  out = mlp_readout(q, q, x1, W1_0, b1s[0], W1_1, b1s[1],
                    W1_2, b1s[2], batch3d)
  return out


# speedup vs baseline: 1.0028x; 1.0010x over previous
import functools

import jax
import jax.numpy as jnp
from jax import lax
from jax.experimental import pallas as pl
from jax.experimental.pallas import tpu as pltpu
from jax.experimental.pallas import tpu_sc as plsc

NC = 2
NS = 16
NW = NC * NS
EW = 128


def _make_sc_segment_sum(n, d, k, n_pad):
  mesh = plsc.VectorSubcoreMesh(core_axis_name="c", subcore_axis_name="s")
  rpt = (-(-n // NS) + 7) // 8 * 8
  rpt_last = n - (NS - 1) * rpt

  nbuf = 2
  iw = 40

  @functools.partial(
      pl.kernel,
      out_type=jax.ShapeDtypeStruct((2 * n, d), jnp.float32),
      mesh=mesh,
      scratch_types=(
          [pltpu.VMEM((iw, EW), jnp.int32),
           pltpu.VMEM((iw, EW), jnp.int32)]
          + [pltpu.VMEM((EW, d), jnp.float32) for _ in range(nbuf)]
          + [pltpu.VMEM_SHARED((n_pad, d), jnp.float32)]
          + [pltpu.SemaphoreType.DMA for _ in range(2 * nbuf + 1)]
      ),
  )
  def seg_sum(x_hbm, ei_hbm, out_hbm, src_v, dst_v,
              r0, r1, acc_sh, g0, g1, s0, s1, seedsem):
    rows = (r0, r1)
    gsem = (g0, g1)
    ssem = (s0, s1)
    c = lax.axis_index("c")
    s = lax.axis_index("s")
    w = c * NS + s
    base = s * rpt

    @pl.when(s < NS - 1)
    def _():
      pltpu.async_copy(x_hbm.at[pl.ds(base, rpt)],
                       acc_sh.at[pl.ds(base, rpt)], seedsem)

    @pl.when(s == NS - 1)
    def _():
      pltpu.async_copy(x_hbm.at[pl.ds(base, rpt_last)],
                       acc_sh.at[pl.ds(base, rpt_last)], seedsem)

    pltpu.sync_copy(ei_hbm.at[0, w, pl.ds(0, iw)], src_v)
    pltpu.sync_copy(ei_hbm.at[1, w, pl.ds(0, iw)], dst_v)
    for b in range(nbuf):
      pltpu.async_copy(x_hbm.at[src_v.at[b]], rows[b], gsem[b])

    @pl.when(s < NS - 1)
    def _():
      pltpu.make_async_copy(x_hbm.at[pl.ds(base, rpt)],
                            acc_sh.at[pl.ds(base, rpt)], seedsem).wait()

    @pl.when(s == NS - 1)
    def _():
      pltpu.make_async_copy(x_hbm.at[pl.ds(base, rpt_last)],
                            acc_sh.at[pl.ds(base, rpt_last)], seedsem).wait()

    plsc.subcore_barrier()

    @pl.loop(0, k // iw)
    def _(wi):
      @pl.when(wi > 0)
      def _():
        pltpu.sync_copy(ei_hbm.at[0, w, pl.ds(wi * iw, iw)], src_v)
        pltpu.sync_copy(ei_hbm.at[1, w, pl.ds(wi * iw, iw)], dst_v)
        for b in range(nbuf):
          pltpu.async_copy(x_hbm.at[src_v.at[b]], rows[b], gsem[b])

      @pl.loop(0, iw // nbuf)
      def _(i):
        j0 = i * nbuf
        for b in range(nbuf):
          pltpu.make_async_copy(x_hbm.at[src_v.at[j0 + b]], rows[b],
                                gsem[b]).wait()
          pltpu.async_copy(rows[b], acc_sh.at[dst_v.at[j0 + b]], ssem[b],
                           add=True)
        for b in range(nbuf):
          pltpu.make_async_copy(rows[b], acc_sh.at[dst_v.at[j0 + b]],
                                ssem[b]).wait()

          @pl.when(j0 + b + nbuf < iw)
          def _():
            pltpu.async_copy(x_hbm.at[src_v.at[j0 + b + nbuf]], rows[b],
                             gsem[b])

    plsc.subcore_barrier()

    @pl.when(s < NS - 1)
    def _():
      pltpu.sync_copy(acc_sh.at[pl.ds(base, rpt)],
                      out_hbm.at[pl.ds(c * n + base, rpt)])

    @pl.when(s == NS - 1)
    def _():
      pltpu.sync_copy(acc_sh.at[pl.ds(base, rpt_last)],
                      out_hbm.at[pl.ds(c * n + base, rpt_last)])

  return seg_sum


def _dot(a, b):
  return lax.dot_general(a, b, (((1,), (0,)), ((), ())),
                         precision=jax.lax.Precision.DEFAULT,
                         preferred_element_type=jnp.float32)


def _mlp_stack(u, w0, b0, w1, b1, w2, b2):
  h = jnp.tanh(_dot(u, w0) + b0)
  h = jnp.tanh(_dot(h, w1) + b1)
  h = jnp.tanh(_dot(h, w2) + b2)
  return jnp.tanh(h)


def _make_mlp(n, d, h, r):
  grid = n // r

  def body(p0_ref, p1_ref, x_ref, w0_ref, b0_ref, w1_ref, b1_ref, w2_ref,
           b2_ref, o_ref):
    u = p0_ref[...] + p1_ref[...] - x_ref[...]
    o_ref[...] = _mlp_stack(u, w0_ref[...], b0_ref[...], w1_ref[...],
                            b1_ref[...], w2_ref[...], b2_ref[...])

  row_spec = pl.BlockSpec((r, d), lambda i: (i, 0))
  return pl.pallas_call(
      body,
      grid=(grid,),
      in_specs=[
          row_spec, pl.BlockSpec((r, d), lambda i: (i + grid, 0)), row_spec,
          pl.BlockSpec((d, h), lambda i: (0, 0)),
          pl.BlockSpec((1, h), lambda i: (0, 0)),
          pl.BlockSpec((h, h), lambda i: (0, 0)),
          pl.BlockSpec((1, h), lambda i: (0, 0)),
          pl.BlockSpec((h, d), lambda i: (0, 0)),
          pl.BlockSpec((1, d), lambda i: (0, 0)),
      ],
      out_specs=row_spec,
      out_shape=jax.ShapeDtypeStruct((n, d), jnp.float32),
  )


def _make_mlp_readout(n, d, h, r, g):
  grid = n // r

  def body(p0_ref, p1_ref, x_ref, w0_ref, b0_ref, w1_ref, b1_ref, w2_ref,
           b2_ref, batch_ref, o_ref, sums_ref, counts_ref):
    i = pl.program_id(0)

    @pl.when(i == 0)
    def _():
      sums_ref[...] = jnp.zeros_like(sums_ref)
      counts_ref[...] = jnp.zeros_like(counts_ref)

    u = p0_ref[...] + p1_ref[...] - x_ref[...]
    xn = _mlp_stack(u, w0_ref[...], b0_ref[...], w1_ref[...], b1_ref[...],
                    w2_ref[...], b2_ref[...])
    gids = lax.broadcasted_iota(jnp.int32, (g, r), 0)
    onehot = (gids == batch_ref[0]).astype(jnp.float32)
    sums_ref[...] += lax.dot_general(
        onehot, xn, (((1,), (0,)), ((), ())),
        precision=jax.lax.Precision.HIGHEST,
        preferred_element_type=jnp.float32)
    cnt = jnp.sum(onehot, axis=1, keepdims=True)
    counts_ref[...] += jnp.broadcast_to(cnt, (g, d))

    @pl.when(i == grid - 1)
    def _():
      o_ref[...] = sums_ref[...] / jnp.maximum(counts_ref[...], 1.0)

  row_spec = pl.BlockSpec((r, d), lambda i: (i, 0))
  return pl.pallas_call(
      body,
      grid=(grid,),
      in_specs=[
          row_spec, pl.BlockSpec((r, d), lambda i: (i + grid, 0)), row_spec,
          pl.BlockSpec((d, h), lambda i: (0, 0)),
          pl.BlockSpec((1, h), lambda i: (0, 0)),
          pl.BlockSpec((h, h), lambda i: (0, 0)),
          pl.BlockSpec((1, h), lambda i: (0, 0)),
          pl.BlockSpec((h, d), lambda i: (0, 0)),
          pl.BlockSpec((1, d), lambda i: (0, 0)),
          pl.BlockSpec((1, 1, r), lambda i: (i, 0, 0)),
      ],
      out_specs=pl.BlockSpec((g, d), lambda i: (0, 0)),
      out_shape=jax.ShapeDtypeStruct((g, d), jnp.float32),
      scratch_shapes=[
          pltpu.VMEM((g, d), jnp.float32),
          pltpu.VMEM((g, d), jnp.float32),
      ],
  )


def kernel(attrs, edge_index, batch,
           W0_0, b0_0, W0_1, b0_1, W0_2, b0_2,
           W1_0, b1_0, W1_1, b1_1, W1_2, b1_2):
  n, d = attrs.shape
  e = edge_index.shape[1]
  h = W0_0.shape[1]
  g = 64
  r = 2000
  k = -(-e // (NW * EW))
  k = -(-k // 40) * 40
  e_pad = NW * k * EW
  n_pad = n + EW

  pad = e_pad - e
  pad_iota = jnp.arange(pad, dtype=jnp.int32) % EW
  ei4d = jnp.concatenate(
      [edge_index, jnp.stack([pad_iota, n + pad_iota])], axis=1
  ).reshape(2, NW, k, EW)
  batch3d = batch.reshape(n // r, 1, r)
  b0s = (b0_0.reshape(1, h), b0_1.reshape(1, h), b0_2.reshape(1, d))
  b1s = (b1_0.reshape(1, h), b1_1.reshape(1, h), b1_2.reshape(1, d))

  seg_sum = _make_sc_segment_sum(n, d, k, n_pad)
  mlp = _make_mlp(n, d, h, r)
  mlp_readout = _make_mlp_readout(n, d, h, r, g)

  p = seg_sum(attrs, ei4d)
  x1 = mlp(p, p, attrs, W0_0, b0s[0], W0_1, b0s[1], W0_2, b0s[2])
  q = seg_sum(x1, ei4d)
  out = mlp_readout(q, q, x1, W1_0, b1s[0], W1_1, b1s[1],
                    W1_2, b1s[2], batch3d)
  return out
